# baseline (speedup 1.0000x reference)
import functools

import jax
import jax.numpy as jnp
from jax import lax
from jax.experimental import pallas as pl
from jax.experimental.pallas import tpu as pltpu
from jax.experimental.pallas import tpu_sc as plsc


def _make_sc_add(b0, S, D):
    info = plsc.get_sparse_core_info()
    NC, NS, L = info.num_cores, info.num_subcores, info.num_lanes
    NW = NC * NS
    rows_per_w = S // NW
    CS = 8
    NR = 2
    n_chunks = rows_per_w // CS
    vecs_per_row = D // L

    mesh = plsc.VectorSubcoreMesh(core_axis_name="c", subcore_axis_name="s")

    @functools.partial(
        pl.kernel,
        mesh=mesh,
        out_type=jax.ShapeDtypeStruct((1, S, D), jnp.float32),
        scratch_types=[
            pltpu.VMEM((NR, CS, D), jnp.float32),
            pltpu.VMEM((2, CS, D), jnp.float32),
        ]
        + [pltpu.SemaphoreType.DMA] * (2 * NR + 2),
    )
    def sc_add(x_hbm, pos_hbm, out_hbm, xbuf, posbuf, *sems):
        ld_sems = sems[:NR]
        st_sems = sems[NR:2 * NR]
        pos_sems = sems[2 * NR:]

        wid = lax.axis_index("s") * NC + lax.axis_index("c")
        base = wid * rows_per_w

        def row0(c):
            return base + c * CS

        def add_chunk(slot, pslot):
            def body(iv, carry):
                col = iv * L
                for r in range(CS):
                    p = posbuf[pslot, r, pl.ds(col, L)]
                    plsc.addupdate(xbuf.at[slot, r, pl.ds(col, L)], p)
                return carry

            lax.fori_loop(0, vecs_per_row, body, None)

        pos_cp = [None, None]
        for c in range(min(2, n_chunks)):
            pos_cp[c] = pltpu.async_copy(
                pos_hbm.at[pl.ds(row0(c), CS)], posbuf.at[c], pos_sems[c]
            )
        load_cp = [None] * n_chunks
        store_cp = [None] * n_chunks

        for i in range(n_chunks + 1):
            if i < n_chunks:
                slot = i % NR
                if i >= NR:
                    store_cp[i - NR].wait()
                load_cp[i] = pltpu.async_copy(
                    x_hbm.at[b0, pl.ds(row0(i), CS)],
                    xbuf.at[slot],
                    ld_sems[slot],
                )
            if i >= 1:
                j = i - 1
                slot = j % NR
                load_cp[j].wait()
                pos_cp[j % 2].wait()
                add_chunk(slot, j % 2)
                if j + 2 < n_chunks:
                    nxt = (j + 2) % 2
                    pos_cp[nxt] = pltpu.async_copy(
                        pos_hbm.at[pl.ds(row0(j + 2), CS)],
                        posbuf.at[nxt],
                        pos_sems[nxt],
                    )
                store_cp[j] = pltpu.async_copy(
                    xbuf.at[slot],
                    out_hbm.at[0, pl.ds(row0(j), CS)],
                    st_sems[slot],
                )

        for j in range(max(0, n_chunks - NR), n_chunks):
            store_cp[j].wait()

    return sc_add


def _tc_body(x_ref, pos_ref, o_ref):
    o_ref[...] = x_ref[...] + pos_ref[...][None, :, :]


def kernel(x, pos_table):
    B, S, D = x.shape
    BS = 512
    B_tc = B - 1

    out_sc = _make_sc_add(B_tc, S, D)(x, pos_table)

    out_tc = pl.pallas_call(
        _tc_body,
        grid=(S // BS, B_tc),
        in_specs=[
            pl.BlockSpec((1, BS, D), lambda s, b: (b, s, 0)),
            pl.BlockSpec((BS, D), lambda s, b: (s, 0)),
        ],
        out_specs=pl.BlockSpec((1, BS, D), lambda s, b: (b, s, 0)),
        out_shape=jax.ShapeDtypeStruct((B_tc, S, D), x.dtype),
        compiler_params=pltpu.CompilerParams(
            dimension_semantics=("arbitrary", "arbitrary"),
        ),
    )(x, pos_table)

    return jnp.concatenate([out_tc, out_sc], axis=0)


# speedup vs baseline: 1.2546x; 1.2546x over previous
import functools

import jax
import jax.numpy as jnp
from jax import lax
from jax.experimental import pallas as pl
from jax.experimental.pallas import tpu as pltpu
from jax.experimental.pallas import tpu_sc as plsc


def _make_sc_add(B, S, D):
    info = plsc.get_sparse_core_info()
    NC, NS, L = info.num_cores, info.num_subcores, info.num_lanes
    NW = NC * NS
    rows_per_w = S // NW
    CS = 8
    NR = 2
    n_chunks = rows_per_w // CS
    vecs_per_row = D // L

    mesh = plsc.VectorSubcoreMesh(core_axis_name="c", subcore_axis_name="s")

    @functools.partial(
        pl.kernel,
        mesh=mesh,
        out_type=jax.ShapeDtypeStruct((B, S, D), jnp.float32),
        scratch_types=[
            pltpu.VMEM((NR, B, CS, D), jnp.float32),
            pltpu.VMEM((2, CS, D), jnp.float32),
        ]
        + [pltpu.SemaphoreType.DMA] * (2 * NR * B + 2),
    )
    def sc_add(x_hbm, pos_hbm, out_hbm, xbuf, posbuf, *sems):
        ld_sems = sems[:NR * B]
        st_sems = sems[NR * B:2 * NR * B]
        pos_sems = sems[2 * NR * B:]

        wid = lax.axis_index("s") * NC + lax.axis_index("c")
        base = wid * rows_per_w

        def row0(c):
            return base + c * CS

        def add_chunk(slot, pslot):
            def body(iv, carry):
                col = iv * L
                for r in range(CS):
                    p = posbuf[pslot, r, pl.ds(col, L)]
                    for b in range(B):
                        plsc.addupdate(xbuf.at[slot, b, r, pl.ds(col, L)], p)
                return carry

            lax.fori_loop(0, vecs_per_row, body, None)

        pos_cp = [None, None]
        for c in range(min(2, n_chunks)):
            pos_cp[c] = pltpu.async_copy(
                pos_hbm.at[pl.ds(row0(c), CS)], posbuf.at[c], pos_sems[c]
            )
        load_cp = [[None] * B for _ in range(n_chunks)]
        store_cp = [[None] * B for _ in range(n_chunks)]

        for i in range(n_chunks + 1):
            if i < n_chunks:
                slot = i % NR
                for b in range(B):
                    if i >= NR:
                        store_cp[i - NR][b].wait()
                    load_cp[i][b] = pltpu.async_copy(
                        x_hbm.at[b, pl.ds(row0(i), CS)],
                        xbuf.at[slot, b],
                        ld_sems[slot * B + b],
                    )
            if i >= 1:
                j = i - 1
                slot = j % NR
                for b in range(B):
                    load_cp[j][b].wait()
                pos_cp[j % 2].wait()
                add_chunk(slot, j % 2)
                if j + 2 < n_chunks:
                    nxt = (j + 2) % 2
                    pos_cp[nxt] = pltpu.async_copy(
                        pos_hbm.at[pl.ds(row0(j + 2), CS)],
                        posbuf.at[nxt],
                        pos_sems[nxt],
                    )
                for b in range(B):
                    store_cp[j][b] = pltpu.async_copy(
                        xbuf.at[slot, b],
                        out_hbm.at[b, pl.ds(row0(j), CS)],
                        st_sems[slot * B + b],
                    )

        for j in range(max(0, n_chunks - NR), n_chunks):
            for b in range(B):
                store_cp[j][b].wait()

    return sc_add


def kernel(x, pos_table):
    B, S, D = x.shape
    return _make_sc_add(B, S, D)(x, pos_table)
